# baseline (speedup 1.0000x reference)
import jax
import jax.numpy as jnp
from jax.experimental import pallas as pl
from jax.experimental.pallas import tpu as pltpu

_NOBJ = 64
_V = 262144
_R = (_V * 3) // 128
_BR = 512
_G = _R // _BR


def _body(obj_sm, x_ref, o_ref):
    o_ref[...] = x_ref[...]


def kernel(obj_id, weights):
    obj = jnp.asarray(obj_id, dtype=jnp.int32)
    s = jnp.where(obj >= 0, jnp.float32(1), jnp.float32(2))
    w = weights.reshape(_NOBJ, _R, 128) * s
    grid_spec = pltpu.PrefetchScalarGridSpec(
        num_scalar_prefetch=1,
        grid=(_G,),
        in_specs=[pl.BlockSpec((1, _BR, 128), lambda i, o: (o[0], i, 0))],
        out_specs=pl.BlockSpec((1, _BR, 128), lambda i, o: (0, i, 0)),
    )
    out_v = pl.pallas_call(
        _body,
        grid_spec=grid_spec,
        out_shape=jax.ShapeDtypeStruct((1, _R, 128), jnp.float32),
    )(obj.reshape(1), w)
    return out_v.reshape(1, _V, 3) * s


# speedup vs baseline: 1.1064x; 1.1064x over previous
import jax
import jax.numpy as jnp
from jax.experimental import pallas as pl
from jax.experimental.pallas import tpu as pltpu

_NOBJ = 64
_V = 262144
_R = (_V * 3) // 128
_BR = 512
_G = _R // _BR


def _body(obj_sm, x_ref, o_ref):
    o_ref[...] = x_ref[...]


def kernel(obj_id, weights):
    obj = jnp.asarray(obj_id, dtype=jnp.int32)
    w = weights.reshape(_NOBJ, _R, 128)
    grid_spec = pltpu.PrefetchScalarGridSpec(
        num_scalar_prefetch=1,
        grid=(_G,),
        in_specs=[pl.BlockSpec((1, _BR, 128), lambda i, o: (o[0], i, 0))],
        out_specs=pl.BlockSpec((1, _BR, 128), lambda i, o: (0, i, 0)),
    )
    out_v = pl.pallas_call(
        _body,
        grid_spec=grid_spec,
        out_shape=jax.ShapeDtypeStruct((1, _R, 128), jnp.float32),
    )(obj.reshape(1), w)
    s = jnp.where(obj >= 0, jnp.float32(1), jnp.float32(2))
    return out_v.reshape(1, _V, 3) * s
